# baseline (speedup 1.0000x reference)
import numpy as np

import jax
import jax.numpy as jnp
from jax.experimental import pallas as pl
from jax.experimental.pallas import tpu as pltpu

_NB = 256
_F1 = 4096
_F2 = 2048


def _onehot_updown(n_in, n_out):
    m = np.zeros((n_in, n_out, 4), np.float32)
    for i in range(n_out):
        for r in range(4):
            h = 2 * i + r
            if h < n_in:
                m[h, i, r] = 1.0
    return m


_IH1 = _onehot_updown(28, 13)
_IH2 = _onehot_updown(13, 5)

_KC4 = np.einsum("hia,wjb->hwijab", _IH1, _IH1).reshape(784, 169 * 16)
_P2 = np.einsum("iAr,jBs->ijABrs", _IH2, _IH2).reshape(169, 25, 16)
_KC5 = np.einsum("ymt,ae->yamte", _P2, np.eye(6, dtype=np.float32)
                 ).reshape(1014, 2400)
_EYE169 = np.eye(169, dtype=np.float32)
_EYE25 = np.eye(25, dtype=np.float32)


def _quad_taps(w_hw):
    parts = []
    for u in (0, 1):
        for v in (0, 1):
            pad = [(0, 0)] * (w_hw.ndim - 2) + [(u, 1 - u), (v, 1 - v)]
            parts.append(jnp.pad(w_hw, pad))
    q = jnp.stack(parts, axis=-1)
    nd = q.ndim
    return jnp.moveaxis(q, (nd - 3, nd - 2), (0, 1))


def _lenet_body(x_ref, w1_ref, b1_ref, w2_ref, b2_ref,
                f1_ref, g1_ref, f2_ref, g2_ref, f3_ref, g3_ref, o_ref):
    f32 = jnp.float32
    bf16 = jnp.bfloat16
    o1 = jnp.dot(x_ref[...].astype(bf16), w1_ref[...],
                 preferred_element_type=f32)
    m1 = jnp.maximum(jnp.maximum(o1[:, 0:1024], o1[:, 1024:2048]),
                     jnp.maximum(o1[:, 2048:3072], o1[:, 3072:4096]))
    m1 = jnp.maximum(m1 + b1_ref[...], 0.0).astype(bf16)
    o2 = jnp.dot(m1, w2_ref[...], preferred_element_type=f32)
    m2 = jnp.maximum(jnp.maximum(o2[:, 0:512], o2[:, 512:1024]),
                     jnp.maximum(o2[:, 1024:1536], o2[:, 1536:2048]))
    m2 = jnp.maximum(m2 + b2_ref[...], 0.0).astype(bf16)
    h1 = jnp.maximum(jnp.dot(m2, f1_ref[...], preferred_element_type=f32)
                     + g1_ref[...], 0.0).astype(bf16)
    h2 = jnp.maximum(jnp.dot(h1, f2_ref[...], preferred_element_type=f32)
                     + g2_ref[...], 0.0).astype(bf16)
    o_ref[...] = jnp.dot(h2, f3_ref[...], preferred_element_type=f32) + g3_ref[...]


def kernel(x, w1, b1, w2, b2, wf1, bf1, wf2, bf2, wf3, bf3):
    f32 = jnp.float32
    bf16 = jnp.bfloat16
    B = x.shape[0]
    nb = _NB if B % _NB == 0 else B
    x2d = x.reshape(B, 784)

    tap1 = jnp.transpose(_quad_taps(w1[:, 0]), (0, 1, 3, 2)
                         ).reshape(16, 4, 6).astype(bf16)
    bd1 = (jnp.asarray(_EYE169, bf16)[:, None, None, :, None]
           * tap1[None, :, :, None, :]).reshape(2704, 4056)
    z1 = jnp.dot(jnp.asarray(_KC4, bf16), bd1,
                 preferred_element_type=f32)
    w1d = jnp.pad(z1.astype(bf16).reshape(784, 4, 1014),
                  ((0, 0), (0, 0), (0, 10))).reshape(784, _F1)
    b1d = jnp.pad(jnp.broadcast_to(b1, (169, 6)).reshape(1, 1014),
                  ((0, 0), (0, 10)))

    tap2 = jnp.transpose(_quad_taps(w2), (0, 1, 3, 4, 2)
                         ).reshape(16, 6, 4, 16).astype(bf16)
    bd2 = (jnp.asarray(_EYE25, bf16)[:, None, None, None, :, None]
           * tap2[None, :, :, :, None, :]).reshape(2400, 1600)
    z2 = jnp.dot(jnp.asarray(_KC5, bf16), bd2,
                 preferred_element_type=f32)
    w2d = jnp.pad(z2.astype(bf16).reshape(1014, 4, 400),
                  ((0, 10), (0, 0), (0, 112))).reshape(1024, _F2)
    b2d = jnp.pad(jnp.broadcast_to(b2, (25, 16)).reshape(1, 400),
                  ((0, 0), (0, 112)))

    wf1p = jnp.pad(wf1.reshape(16, 5, 5, 120).transpose(1, 2, 0, 3).reshape(400, 120),
                   ((0, 112), (0, 8))).astype(bf16)
    bf1p = jnp.pad(bf1, (0, 8)).reshape(1, 128)
    wf2p = jnp.pad(wf2, ((0, 8), (0, 44))).astype(bf16)
    bf2p = jnp.pad(bf2, (0, 44)).reshape(1, 128)
    wf3p = jnp.pad(wf3, ((0, 44), (0, 118))).astype(bf16)
    bf3p = jnp.pad(bf3, (0, 118)).reshape(1, 128)

    const = lambda: (lambda b: (0, 0))
    out = pl.pallas_call(
        _lenet_body,
        out_shape=jax.ShapeDtypeStruct((B, 128), f32),
        grid=(B // nb,),
        in_specs=[
            pl.BlockSpec((nb, 784), lambda b: (b, 0)),
            pl.BlockSpec((784, _F1), const()),
            pl.BlockSpec((1, 1024), const()),
            pl.BlockSpec((1024, _F2), const()),
            pl.BlockSpec((1, 512), const()),
            pl.BlockSpec((512, 128), const()),
            pl.BlockSpec((1, 128), const()),
            pl.BlockSpec((128, 128), const()),
            pl.BlockSpec((1, 128), const()),
            pl.BlockSpec((128, 128), const()),
            pl.BlockSpec((1, 128), const()),
        ],
        out_specs=pl.BlockSpec((nb, 128), lambda b: (b, 0)),
        compiler_params=pltpu.CompilerParams(
            dimension_semantics=("parallel",),
            vmem_limit_bytes=100 * 1024 * 1024,
        ),
    )(x2d, w1d, b1d, w2d, b2d, wf1p, bf1p, wf2p, bf2p, wf3p, bf3p)
    return out[:, :10]


# speedup vs baseline: 2.3249x; 2.3249x over previous
import numpy as np

import jax
import jax.numpy as jnp
from jax.experimental import pallas as pl
from jax.experimental.pallas import tpu as pltpu

_NB = 256
_F1 = 4096
_F2 = 2048


def _onehot_updown(n_in, n_out):
    m = np.zeros((n_in, n_out, 4), np.float32)
    for i in range(n_out):
        for r in range(4):
            h = 2 * i + r
            if h < n_in:
                m[h, i, r] = 1.0
    return m


_IH1 = _onehot_updown(28, 13)
_IH2 = _onehot_updown(13, 5)

_KC4 = np.einsum("hia,wjb->hwijab", _IH1, _IH1).reshape(784, 169 * 16)
_P2 = np.einsum("iAr,jBs->ijABrs", _IH2, _IH2).reshape(169, 25, 16)
_KC5 = np.einsum("ymt,ae->yamte", _P2, np.eye(6, dtype=np.float32)
                 ).reshape(1014, 2400)
_EYE1 = np.kron(np.eye(169, dtype=np.float32),
                np.ones((16, 6), np.float32))
_EYE2 = np.kron(np.eye(25, dtype=np.float32),
                np.ones((96, 16), np.float32))


def _quad_taps(w_hw):
    parts = []
    for u in (0, 1):
        for v in (0, 1):
            pad = [(0, 0)] * (w_hw.ndim - 2) + [(u, 1 - u), (v, 1 - v)]
            parts.append(jnp.pad(w_hw, pad))
    q = jnp.stack(parts, axis=-1)
    nd = q.ndim
    return jnp.moveaxis(q, (nd - 3, nd - 2), (0, 1))


def _build_body(t1_ref, t2_ref, kc4_ref, kc5_ref, e1_ref, e2_ref,
                w1d_ref, w2d_ref):
    f32 = jnp.float32
    bf16 = jnp.bfloat16
    kc4 = kc4_ref[...]
    kc5 = kc5_ref[...]
    e1 = e1_ref[...]
    e2 = e2_ref[...]
    t1b = t1_ref[...].astype(bf16)
    t2b = t2_ref[...].astype(bf16)
    zpad1 = jnp.zeros((784, 10), bf16)
    pieces1 = []
    pieces2 = []
    for q in range(4):
        tap1 = t1b[:, q * 6:(q + 1) * 6]
        tapb1 = pltpu.repeat(pltpu.repeat(tap1, 169, axis=0),
                             169, axis=1)
        bd1 = tapb1 * e1
        z1 = jnp.dot(kc4, bd1, preferred_element_type=f32)
        pieces1 += [z1.astype(bf16), zpad1]

        tap2 = t2b[:, q * 16:(q + 1) * 16]
        tapb2 = pltpu.repeat(pltpu.repeat(tap2, 25, axis=0),
                             25, axis=1)
        bd2 = tapb2 * e2
        z2 = jnp.dot(kc5, bd2, preferred_element_type=f32)
        pieces2 += [z2.astype(bf16), jnp.zeros((1014, 112), bf16)]
    w1d_ref[...] = jnp.concatenate(pieces1, axis=1)
    w2d_ref[...] = jnp.concatenate(
        [jnp.concatenate(pieces2, axis=1),
         jnp.zeros((10, 2048), bf16)], axis=0)


def _lenet_body(x_ref, w1_ref, b1_ref, w2_ref, b2_ref,
                f1_ref, g1_ref, f2_ref, g2_ref, f3_ref, g3_ref, o_ref):
    f32 = jnp.float32
    bf16 = jnp.bfloat16
    o1 = jnp.dot(x_ref[...].astype(bf16), w1_ref[...],
                 preferred_element_type=f32)
    m1 = jnp.maximum(jnp.maximum(o1[:, 0:1024], o1[:, 1024:2048]),
                     jnp.maximum(o1[:, 2048:3072], o1[:, 3072:4096]))
    m1 = jnp.maximum(m1 + b1_ref[...], 0.0).astype(bf16)
    o2 = jnp.dot(m1, w2_ref[...], preferred_element_type=f32)
    m2 = jnp.maximum(jnp.maximum(o2[:, 0:512], o2[:, 512:1024]),
                     jnp.maximum(o2[:, 1024:1536], o2[:, 1536:2048]))
    m2 = jnp.maximum(m2 + b2_ref[...], 0.0).astype(bf16)
    h1 = jnp.maximum(jnp.dot(m2, f1_ref[...], preferred_element_type=f32)
                     + g1_ref[...], 0.0).astype(bf16)
    h2 = jnp.maximum(jnp.dot(h1, f2_ref[...], preferred_element_type=f32)
                     + g2_ref[...], 0.0).astype(bf16)
    o_ref[...] = jnp.dot(h2, f3_ref[...], preferred_element_type=f32) + g3_ref[...]


def kernel(x, w1, b1, w2, b2, wf1, bf1, wf2, bf2, wf3, bf3):
    f32 = jnp.float32
    bf16 = jnp.bfloat16
    B = x.shape[0]
    nb = _NB if B % _NB == 0 else B
    x2d = x.reshape(B, 784)

    t1 = jnp.transpose(_quad_taps(w1[:, 0]), (0, 1, 3, 2)).reshape(16, 24)
    t2 = jnp.transpose(_quad_taps(w2), (0, 1, 3, 4, 2)).reshape(96, 64)

    wb = lambda: (lambda: (0, 0))
    w1d, w2d = pl.pallas_call(
        _build_body,
        out_shape=(jax.ShapeDtypeStruct((784, _F1), bf16),
                   jax.ShapeDtypeStruct((1024, _F2), bf16)),
        compiler_params=pltpu.CompilerParams(
            vmem_limit_bytes=100 * 1024 * 1024,
        ),
    )(t1, t2, jnp.asarray(_KC4, bf16), jnp.asarray(_KC5, bf16),
      jnp.asarray(_EYE1, bf16), jnp.asarray(_EYE2, bf16))

    b1d = jnp.pad(jnp.broadcast_to(b1, (169, 6)).reshape(1, 1014),
                  ((0, 0), (0, 10)))
    b2d = jnp.pad(jnp.broadcast_to(b2, (25, 16)).reshape(1, 400),
                  ((0, 0), (0, 112)))

    wf1p = jnp.pad(wf1.reshape(16, 5, 5, 120).transpose(1, 2, 0, 3).reshape(400, 120),
                   ((0, 112), (0, 8))).astype(bf16)
    bf1p = jnp.pad(bf1, (0, 8)).reshape(1, 128)
    wf2p = jnp.pad(wf2, ((0, 8), (0, 44))).astype(bf16)
    bf2p = jnp.pad(bf2, (0, 44)).reshape(1, 128)
    wf3p = jnp.pad(wf3, ((0, 44), (0, 118))).astype(bf16)
    bf3p = jnp.pad(bf3, (0, 118)).reshape(1, 128)

    const = lambda: (lambda b: (0, 0))
    out = pl.pallas_call(
        _lenet_body,
        out_shape=jax.ShapeDtypeStruct((B, 128), f32),
        grid=(B // nb,),
        in_specs=[
            pl.BlockSpec((nb, 784), lambda b: (b, 0)),
            pl.BlockSpec((784, _F1), const()),
            pl.BlockSpec((1, 1024), const()),
            pl.BlockSpec((1024, _F2), const()),
            pl.BlockSpec((1, 512), const()),
            pl.BlockSpec((512, 128), const()),
            pl.BlockSpec((1, 128), const()),
            pl.BlockSpec((128, 128), const()),
            pl.BlockSpec((1, 128), const()),
            pl.BlockSpec((128, 128), const()),
            pl.BlockSpec((1, 128), const()),
        ],
        out_specs=pl.BlockSpec((nb, 128), lambda b: (b, 0)),
        compiler_params=pltpu.CompilerParams(
            dimension_semantics=("parallel",),
            vmem_limit_bytes=100 * 1024 * 1024,
        ),
    )(x2d, w1d, b1d, w2d, b2d, wf1p, bf1p, wf2p, bf2p, wf3p, bf3p)
    return out[:, :10]
